# baseline (speedup 1.0000x reference)
import functools

import jax
import jax.numpy as jnp
from jax import lax
from jax.experimental import pallas as pl
from jax.experimental.pallas import tpu as pltpu
from jax.experimental.pallas import tpu_sc as plsc

HEADS = 8
DIM = 192
DH = 64
NB = 2
N = 4096
BH = NB * HEADS
NSEL = 512
F32 = jnp.float32


def _prep_kernel(x_ref, g_ref, b_ref, w_ref,
                 qn_ref, kv_ref, sc_ref, xn_scr):
    h = pl.program_id(1)

    @pl.when(h == 0)
    def _():
        xb = x_ref[0]
        mu = jnp.mean(xb, axis=1, keepdims=True)
        xc = xb - mu
        var = jnp.mean(xc * xc, axis=1, keepdims=True)
        inv = 1.0 / (jnp.sqrt(var) + 1e-6)
        xn_scr[...] = (xc * inv) * g_ref[...] + b_ref[...]

    xn = xn_scr[...]
    nt = (((1,), (1,)), ((), ()))
    qkv = lax.dot_general(xn, w_ref[0], nt, preferred_element_type=F32)
    q = qkv[:, :DH]
    k = qkv[:, DH:2 * DH]
    v = qkv[:, 2 * DH:]
    qi = 1.0 / (jnp.sqrt(jnp.sum(q * q, axis=1, keepdims=True)) + 1e-6)
    ki = 1.0 / (jnp.sqrt(jnp.sum(k * k, axis=1, keepdims=True)) + 1e-6)
    qn = q * qi
    kn = k * ki

    qn_ref[0, 0] = qn
    kv_ref[0, 0] = jnp.concatenate([kn, v], axis=1)

    q_probe = jnp.sum(jnp.abs(qn), axis=0, keepdims=True)
    ka = jnp.abs(kn)
    kd = jnp.sum(ka.reshape(16, 256, DH), axis=1)
    k4 = ka.reshape(16, 16, 16, DH)
    kh = jnp.sum(jnp.sum(k4, axis=0), axis=1)
    kw = jnp.sum(ka.reshape(256, 16, DH), axis=0)
    bf = jnp.bfloat16
    qpb = q_probe.astype(bf).astype(F32)
    s_d = jnp.sum(kd.astype(bf).astype(F32) * qpb, axis=1, keepdims=True)
    s_h = jnp.sum(kh.astype(bf).astype(F32) * qpb, axis=1, keepdims=True)
    s_w = jnp.sum(kw.astype(bf).astype(F32) * qpb, axis=1, keepdims=True)

    ii_col = lax.broadcasted_iota(jnp.int32, (16, 1), 0).astype(F32)
    parts = []
    for a, s_col in enumerate((s_d, s_h, s_w)):
        vv = s_col
        vals = []
        for _ in range(8):
            mval = jnp.max(vv, axis=0, keepdims=True)
            mi = jnp.where(vv == mval, ii_col, 3e38)
            mn = jnp.min(mi, axis=0, keepdims=True)
            vals.append(mn)
            vv = jnp.where(mi == mn, -3e38, vv)
        if a == 0:
            parts.append(jnp.concatenate(vals, axis=0))
        else:
            parts.append(jnp.concatenate(vals, axis=1))
    td_col, th_row, tw_row = parts

    e8 = lax.broadcasted_iota(jnp.int32, (8, 64), 0)
    c64 = lax.broadcasted_iota(jnp.int32, (8, 64), 1)
    expand_h = (c64 // 8 == e8).astype(F32)
    expand_w = (c64 % 8 == e8).astype(F32)
    th_e = lax.dot_general(th_row, expand_h, (((1,), (0,)), ((), ())),
                           preferred_element_type=F32)
    tw_e = lax.dot_general(tw_row, expand_w, (((1,), (0,)), ((), ())),
                           preferred_element_type=F32)
    bh = pl.program_id(0) * HEADS + pl.program_id(1)
    base = (bh * N).astype(F32)
    m = td_col * 256.0 + th_e * 16.0 + tw_e + base
    sc_ref[0, 0] = m.astype(jnp.int32)


@functools.cache
def _sc_gather_fn():
    mesh = plsc.VectorSubcoreMesh(
        core_axis_name="c", subcore_axis_name="s", num_cores=2,
        num_subcores=16)
    return functools.partial(
        pl.kernel,
        out_type=jax.ShapeDtypeStruct((BH * NSEL, 2 * DH), F32),
        mesh=mesh,
        scratch_types=[
            pltpu.VMEM((128,), jnp.int32),
            pltpu.VMEM((128,), jnp.int32),
            pltpu.VMEM((256, 2 * DH), F32),
            pltpu.SemaphoreType.DMA,
        ],
    )(_sc_gather)


def _sc_gather(idx_hbm, kv_hbm, out_hbm, idx_a, idx_b, rows_v, sem):
    cid = lax.axis_index("c")
    sid = lax.axis_index("s")
    wid = sid * 2 + cid
    bh = wid // 2
    half = wid % 2
    off = bh * NSEL + half * 256

    pltpu.sync_copy(idx_hbm.at[pl.ds(off, 128)], idx_a)
    pltpu.sync_copy(idx_hbm.at[pl.ds(off + 128, 128)], idx_b)
    cp0 = pltpu.async_copy(kv_hbm.at[idx_a], rows_v.at[pl.ds(0, 128)], sem)
    cp1 = pltpu.async_copy(kv_hbm.at[idx_b], rows_v.at[pl.ds(128, 128)], sem)
    cp0.wait()
    cp1.wait()
    pltpu.sync_copy(rows_v, out_hbm.at[pl.ds(off, 256)])


def _attn_kernel(qn_ref, k_ref, v_ref, wo_ref, bo_ref, out_ref):
    h = pl.program_id(1)
    q = qn_ref[0, 0]
    k = k_ref[0]
    v = v_ref[0]
    sim = lax.dot_general(q, k, (((1,), (1,)), ((), ())),
                          preferred_element_type=F32)
    p = jnp.exp(sim)
    s = jnp.sum(p, axis=1, keepdims=True)
    o = lax.dot_general(p, v, (((1,), (0,)), ((), ())),
                        preferred_element_type=F32)
    o = o * (1.0 / s)
    part = lax.dot_general(wo_ref[0], o, (((1,), (1,)), ((), ())),
                           preferred_element_type=F32)

    @pl.when(h == 0)
    def _():
        out_ref[0] = part + bo_ref[...]

    @pl.when(h != 0)
    def _():
        out_ref[0] += part


@jax.jit
def kernel(x, gamma, beta, W_qkv, W_out, b_out):
    b, c, D, H, W = x.shape
    xt = x.reshape(NB, DIM, N).transpose(0, 2, 1)
    g2 = gamma.reshape(1, DIM)
    b2 = beta.reshape(1, DIM)
    w3 = W_qkv.reshape(3, HEADS, DH, DIM)
    wcat = w3.transpose(1, 0, 2, 3).reshape(HEADS, 3 * DH, DIM)
    wo = W_out.reshape(DH, HEADS, DH).transpose(1, 0, 2)
    bo = b_out.reshape(DH, 1)

    qn, kv, idx = pl.pallas_call(
        _prep_kernel,
        grid=(NB, HEADS),
        in_specs=[
            pl.BlockSpec((1, N, DIM), lambda b_, h_: (b_, 0, 0)),
            pl.BlockSpec((1, DIM), lambda b_, h_: (0, 0)),
            pl.BlockSpec((1, DIM), lambda b_, h_: (0, 0)),
            pl.BlockSpec((1, 3 * DH, DIM), lambda b_, h_: (h_, 0, 0)),
        ],
        out_specs=[
            pl.BlockSpec((1, 1, N, DH), lambda b_, h_: (b_, h_, 0, 0)),
            pl.BlockSpec((1, 1, N, 2 * DH), lambda b_, h_: (b_, h_, 0, 0)),
            pl.BlockSpec((1, 1, 8, 64), lambda b_, h_: (b_, h_, 0, 0)),
        ],
        out_shape=[
            jax.ShapeDtypeStruct((NB, HEADS, N, DH), F32),
            jax.ShapeDtypeStruct((NB, HEADS, N, 2 * DH), F32),
            jax.ShapeDtypeStruct((NB, HEADS, 8, 64), jnp.int32),
        ],
        scratch_shapes=[pltpu.VMEM((N, DIM), F32)],
    )(xt, g2, b2, wcat)

    kv_flat = kv.reshape(BH * N, 2 * DH)
    idx_flat = idx.reshape(BH * NSEL)
    kvsel = _sc_gather_fn()(idx_flat, kv_flat)
    kvsel = kvsel.reshape(BH, NSEL, 2 * DH)
    ksel = kvsel[:, :, :DH]
    vsel = kvsel[:, :, DH:]

    out = pl.pallas_call(
        _attn_kernel,
        grid=(NB, HEADS),
        in_specs=[
            pl.BlockSpec((1, 1, N, DH), lambda b_, h_: (b_, h_, 0, 0)),
            pl.BlockSpec((1, NSEL, DH), lambda b_, h_: (b_ * HEADS + h_, 0, 0)),
            pl.BlockSpec((1, NSEL, DH), lambda b_, h_: (b_ * HEADS + h_, 0, 0)),
            pl.BlockSpec((1, DH, DH), lambda b_, h_: (h_, 0, 0)),
            pl.BlockSpec((DH, 1), lambda b_, h_: (0, 0)),
        ],
        out_specs=pl.BlockSpec((1, DH, N), lambda b_, h_: (b_, 0, 0)),
        out_shape=jax.ShapeDtypeStruct((NB, DH, N), F32),
    )(qn, ksel, vsel, wo, bo)

    return out.reshape(NB, DH, D, H, W)


# speedup vs baseline: 1.0457x; 1.0457x over previous
import functools

import jax
import jax.numpy as jnp
from jax import lax
from jax.experimental import pallas as pl
from jax.experimental.pallas import tpu as pltpu
from jax.experimental.pallas import tpu_sc as plsc

HEADS = 8
DIM = 192
DH = 64
NB = 2
N = 4096
BH = NB * HEADS
NSEL = 512
F32 = jnp.float32


def _prep_kernel(x_ref, g_ref, b_ref, w_ref,
                 qn_ref, kv_ref, sc_ref, xn_scr):
    h = pl.program_id(1)

    @pl.when(h == 0)
    def _():
        xb = x_ref[0]
        mu = jnp.mean(xb, axis=1, keepdims=True)
        xc = xb - mu
        var = jnp.mean(xc * xc, axis=1, keepdims=True)
        xn_scr[...] = g_ref[...] * xc / (jnp.sqrt(var) + 1e-6) + b_ref[...]

    xn = xn_scr[...]
    nt = (((1,), (1,)), ((), ()))
    qkv = lax.dot_general(xn, w_ref[0], nt, preferred_element_type=F32)
    q = qkv[:, :DH]
    k = qkv[:, DH:2 * DH]
    v = qkv[:, 2 * DH:]
    qn = q / (jnp.sqrt(jnp.sum(q * q, axis=1, keepdims=True)) + 1e-6)
    kn = k / (jnp.sqrt(jnp.sum(k * k, axis=1, keepdims=True)) + 1e-6)

    qn_ref[0, 0] = qn
    kv_ref[0, 0] = jnp.concatenate([kn, v], axis=1)

    q_probe = jnp.sum(jnp.abs(qn), axis=0, keepdims=True)
    ka = jnp.abs(kn)
    kd = jnp.sum(ka.reshape(16, 256, DH), axis=1)
    k4 = ka.reshape(16, 16, 16, DH)
    kh = jnp.sum(jnp.sum(k4, axis=0), axis=1)
    kw = jnp.sum(ka.reshape(256, 16, DH), axis=0)
    bf = jnp.bfloat16
    qpb = q_probe.astype(bf).astype(F32)
    s_d = jnp.sum(kd.astype(bf).astype(F32) * qpb, axis=1, keepdims=True)
    s_h = jnp.sum(kh.astype(bf).astype(F32) * qpb, axis=1, keepdims=True)
    s_w = jnp.sum(kw.astype(bf).astype(F32) * qpb, axis=1, keepdims=True)

    ii_col = lax.broadcasted_iota(jnp.int32, (16, 1), 0).astype(F32)
    parts = []
    for a, s_col in enumerate((s_d, s_h, s_w)):
        vv = s_col
        vals = []
        for _ in range(8):
            mval = jnp.max(vv, axis=0, keepdims=True)
            mi = jnp.where(vv == mval, ii_col, 3e38)
            mn = jnp.min(mi, axis=0, keepdims=True)
            vals.append(mn)
            vv = jnp.where(mi == mn, -3e38, vv)
        if a == 0:
            parts.append(jnp.concatenate(vals, axis=0))
        else:
            parts.append(jnp.concatenate(vals, axis=1))
    td_col, th_row, tw_row = parts

    e8 = lax.broadcasted_iota(jnp.int32, (8, 64), 0)
    c64 = lax.broadcasted_iota(jnp.int32, (8, 64), 1)
    expand_h = (c64 // 8 == e8).astype(F32)
    expand_w = (c64 % 8 == e8).astype(F32)
    th_e = lax.dot_general(th_row, expand_h, (((1,), (0,)), ((), ())),
                           preferred_element_type=F32)
    tw_e = lax.dot_general(tw_row, expand_w, (((1,), (0,)), ((), ())),
                           preferred_element_type=F32)
    bh = pl.program_id(0) * HEADS + pl.program_id(1)
    base = (bh * N).astype(F32)
    m = td_col * 256.0 + th_e * 16.0 + tw_e + base
    sc_ref[0, 0] = m.astype(jnp.int32)


@functools.cache
def _sc_gather_fn():
    mesh = plsc.VectorSubcoreMesh(
        core_axis_name="c", subcore_axis_name="s", num_cores=2,
        num_subcores=16)
    return functools.partial(
        pl.kernel,
        out_type=jax.ShapeDtypeStruct((BH * NSEL, 2 * DH), F32),
        mesh=mesh,
        scratch_types=[
            pltpu.VMEM((128,), jnp.int32),
            pltpu.VMEM((128,), jnp.int32),
            pltpu.VMEM((256, 2 * DH), F32),
            pltpu.SemaphoreType.DMA,
        ],
    )(_sc_gather)


def _sc_gather(idx_hbm, kv_hbm, out_hbm, idx_a, idx_b, rows_v, sem):
    cid = lax.axis_index("c")
    sid = lax.axis_index("s")
    wid = sid * 2 + cid
    bh = wid // 2
    half = wid % 2
    off = bh * NSEL + half * 256

    pltpu.sync_copy(idx_hbm.at[pl.ds(off, 128)], idx_a)
    pltpu.sync_copy(idx_hbm.at[pl.ds(off + 128, 128)], idx_b)
    cp0 = pltpu.async_copy(kv_hbm.at[idx_a], rows_v.at[pl.ds(0, 128)], sem)
    cp1 = pltpu.async_copy(kv_hbm.at[idx_b], rows_v.at[pl.ds(128, 128)], sem)
    cp0.wait()
    cp1.wait()
    pltpu.sync_copy(rows_v, out_hbm.at[pl.ds(off, 256)])


def _attn_kernel(qn_ref, kv_ref, wo_ref, bo_ref, out_ref):
    h = pl.program_id(1)
    q = qn_ref[0, 0]
    k = kv_ref[0, :, :DH]
    v = kv_ref[0, :, DH:]
    sim = lax.dot_general(q, k, (((1,), (1,)), ((), ())),
                          preferred_element_type=F32)
    p = jnp.exp(sim)
    s = jnp.sum(p, axis=1, keepdims=True)
    o = lax.dot_general(p, v, (((1,), (0,)), ((), ())),
                        preferred_element_type=F32)
    o = o / s
    part = lax.dot_general(wo_ref[0], o, (((1,), (1,)), ((), ())),
                           preferred_element_type=F32)

    @pl.when(h == 0)
    def _():
        out_ref[0] = part + bo_ref[...]

    @pl.when(h != 0)
    def _():
        out_ref[0] += part


@jax.jit
def kernel(x, gamma, beta, W_qkv, W_out, b_out):
    b, c, D, H, W = x.shape
    xt = x.reshape(NB, DIM, N).transpose(0, 2, 1)
    g2 = gamma.reshape(1, DIM)
    b2 = beta.reshape(1, DIM)
    w3 = W_qkv.reshape(3, HEADS, DH, DIM)
    wcat = w3.transpose(1, 0, 2, 3).reshape(HEADS, 3 * DH, DIM)
    wo = W_out.reshape(DH, HEADS, DH).transpose(1, 0, 2)
    bo = b_out.reshape(DH, 1)

    qn, kv, idx = pl.pallas_call(
        _prep_kernel,
        grid=(NB, HEADS),
        in_specs=[
            pl.BlockSpec((1, N, DIM), lambda b_, h_: (b_, 0, 0)),
            pl.BlockSpec((1, DIM), lambda b_, h_: (0, 0)),
            pl.BlockSpec((1, DIM), lambda b_, h_: (0, 0)),
            pl.BlockSpec((1, 3 * DH, DIM), lambda b_, h_: (h_, 0, 0)),
        ],
        out_specs=[
            pl.BlockSpec((1, 1, N, DH), lambda b_, h_: (b_, h_, 0, 0)),
            pl.BlockSpec((1, 1, N, 2 * DH), lambda b_, h_: (b_, h_, 0, 0)),
            pl.BlockSpec((1, 1, 8, 64), lambda b_, h_: (b_, h_, 0, 0)),
        ],
        out_shape=[
            jax.ShapeDtypeStruct((NB, HEADS, N, DH), F32),
            jax.ShapeDtypeStruct((NB, HEADS, N, 2 * DH), F32),
            jax.ShapeDtypeStruct((NB, HEADS, 8, 64), jnp.int32),
        ],
        scratch_shapes=[pltpu.VMEM((N, DIM), F32)],
    )(xt, g2, b2, wcat)

    kv_flat = kv.reshape(BH * N, 2 * DH)
    idx_flat = idx.reshape(BH * NSEL)
    kvsel = _sc_gather_fn()(idx_flat, kv_flat)
    kvsel = kvsel.reshape(BH, NSEL, 2 * DH)

    out = pl.pallas_call(
        _attn_kernel,
        grid=(NB, HEADS),
        in_specs=[
            pl.BlockSpec((1, 1, N, DH), lambda b_, h_: (b_, h_, 0, 0)),
            pl.BlockSpec((1, NSEL, 2 * DH), lambda b_, h_: (b_ * HEADS + h_, 0, 0)),
            pl.BlockSpec((1, DH, DH), lambda b_, h_: (h_, 0, 0)),
            pl.BlockSpec((DH, 1), lambda b_, h_: (0, 0)),
        ],
        out_specs=pl.BlockSpec((1, DH, N), lambda b_, h_: (b_, 0, 0)),
        out_shape=jax.ShapeDtypeStruct((NB, DH, N), F32),
    )(qn, kvsel, wo, bo)

    return out.reshape(NB, DH, D, H, W)
